# baseline (speedup 1.0000x reference)
import functools

import jax
import jax.numpy as jnp
from jax import lax
from jax.experimental import pallas as pl
from jax.experimental.pallas import tpu as pltpu
from jax.experimental.pallas import tpu_sc as plsc

_N_AMINO = 30
_N_STRUCT = 8
_D = 128
_B = 1024
_L = 200
_NW = 32
_ROWS_PER_W = _B // _NW
_NJ = _D // 16
_LPAD = _L + 16
_EPS = 1e-5


def _rsqrt(a):
    a0 = a[0]
    i = lax.bitcast_convert_type(a0, jnp.int32)
    i = jnp.int32(0x5F3759DF) - lax.shift_right_logical(i, jnp.int32(1))
    y = jnp.broadcast_to(lax.bitcast_convert_type(i, jnp.float32), (16,))
    h = a * jnp.float32(-0.5)
    for _ in range(3):
        y = y * (h * (y * y) + jnp.float32(1.5))
    return y


def _lane_sum(v, perms):
    for p in perms:
        v = v + v.at[p].get(mode="promise_in_bounds")
    return v


def _body(aseq, sseq, atab, stab, ptab, gam, bet, out,
          aidx_v, sidx_v, pos_v, comb_v, g_v, b_v, bufd, semd):
    wid = lax.axis_index("s") * 2 + lax.axis_index("c")
    r0 = wid * _ROWS_PER_W

    pltpu.sync_copy(aseq.at[pl.ds(r0, _ROWS_PER_W)], aidx_v)
    pltpu.sync_copy(sseq.at[pl.ds(r0, _ROWS_PER_W)], sidx_v)
    pltpu.sync_copy(ptab.at[pl.ds(0, _L)], pos_v)
    pltpu.sync_copy(gam, g_v)
    pltpu.sync_copy(bet, b_v)
    stage = bufd.at[0]
    pltpu.sync_copy(atab, stage.at[pl.ds(0, _N_AMINO)])
    pltpu.sync_copy(stab, stage.at[pl.ds(32, _N_STRUCT)])

    def _build(ci, carry):
        ai = lax.shift_right_logical(ci, jnp.int32(3))
        si = lax.bitwise_and(ci, jnp.int32(_N_STRUCT - 1))
        for j in range(_NJ):
            a = stage[ai, pl.ds(j * 16, 16)]
            s = stage[si + 32, pl.ds(j * 16, 16)]
            comb_v[ci, pl.ds(j * 16, 16)] = a + s
        return carry

    lax.fori_loop(jnp.int32(0), jnp.int32(_N_AMINO * _N_STRUCT), _build, 0)

    idx16 = lax.iota(jnp.int32, 16)
    perms = [idx16 ^ jnp.int32(1 << k) for k in range(4)]
    g = [g_v[pl.ds(j * 16, 16)] for j in range(_NJ)]
    b = [b_v[pl.ds(j * 16, 16)] for j in range(_NJ)]
    inv_d = jnp.float32(1.0 / _D)

    def _one_token(l, civ, k, buf):
        ci = civ[k]
        x = [comb_v[ci, pl.ds(j * 16, 16)] + pos_v[l, pl.ds(j * 16, 16)]
             for j in range(_NJ)]
        s = ((x[0] + x[1]) + (x[2] + x[3])) + ((x[4] + x[5]) + (x[6] + x[7]))
        s = _lane_sum(s, perms)
        sq = [xj * xj for xj in x]
        q = ((sq[0] + sq[1]) + (sq[2] + sq[3])) + ((sq[4] + sq[5]) + (sq[6] + sq[7]))
        q = _lane_sum(q, perms)
        mean = s * inv_d
        var = q * inv_d - mean * mean
        rstd = _rsqrt(var + jnp.float32(_EPS))
        c = jnp.float32(0.0) - mean * rstd
        for j in range(_NJ):
            y = x[j] * rstd + c
            buf[l, pl.ds(j * 16, 16)] = y * g[j] + b[j]

    def _row(r, carry):
        par = lax.bitwise_and(r, jnp.int32(1))
        buf = bufd.at[par]
        sem = semd.at[par]

        @pl.when(r >= 2)
        def _drain():
            pltpu.make_async_copy(buf, out.at[r0], sem).wait()

        def _group(gi, carry2):
            av = aidx_v[r, pl.ds(gi * 16, 16)]
            sv = sidx_v[r, pl.ds(gi * 16, 16)]
            civ = av * jnp.int32(_N_STRUCT) + sv
            for k in range(16):
                _one_token(gi * 16 + k, civ, k, buf)
            return carry2

        lax.fori_loop(jnp.int32(0), jnp.int32(_L // 16), _group, 0)
        tail0 = r * jnp.int32(0) + jnp.int32(_L - 8)
        av = aidx_v[r, pl.ds(tail0, 16)]
        sv = sidx_v[r, pl.ds(tail0, 16)]
        civ = av * jnp.int32(_N_STRUCT) + sv
        for k in range(8):
            _one_token(jnp.int32(_L - 8 + k), civ, k, buf)
        pltpu.async_copy(buf, out.at[r0 + r], sem)
        return carry

    lax.fori_loop(jnp.int32(0), jnp.int32(_ROWS_PER_W), _row, 0)
    pltpu.make_async_copy(bufd.at[0], out.at[r0], semd.at[0]).wait()
    pltpu.make_async_copy(bufd.at[1], out.at[r0], semd.at[1]).wait()


_sc_kernel = functools.partial(
    pl.kernel,
    out_type=jax.ShapeDtypeStruct((_B, _L, _D), jnp.float32),
    mesh=plsc.VectorSubcoreMesh(core_axis_name="c", subcore_axis_name="s"),
    scratch_types=[
        pltpu.VMEM((_ROWS_PER_W, _L), jnp.int32),
        pltpu.VMEM((_ROWS_PER_W, _L), jnp.int32),
        pltpu.VMEM((_L, _D), jnp.float32),
        pltpu.VMEM((_N_AMINO * _N_STRUCT, _D), jnp.float32),
        pltpu.VMEM((_D,), jnp.float32),
        pltpu.VMEM((_D,), jnp.float32),
        pltpu.VMEM((2, _L, _D), jnp.float32),
        pltpu.SemaphoreType.DMA((2,)),
    ],
)(_body)


def kernel(amino_seq, struct_seq, amino_table, struct_table, pos_table, gamma, beta):
    return _sc_kernel(amino_seq, struct_seq, amino_table, struct_table,
                      pos_table, gamma, beta)


# speedup vs baseline: 1.3691x; 1.3691x over previous
import functools

import jax
import jax.numpy as jnp
from jax import lax
from jax.experimental import pallas as pl
from jax.experimental.pallas import tpu as pltpu
from jax.experimental.pallas import tpu_sc as plsc

_N_AMINO = 30
_N_STRUCT = 8
_D = 128
_B = 1024
_L = 200
_NW = 32
_ROWS_PER_W = _B // _NW
_NJ = _D // 16
_LPAD = _L + 16
_EPS = 1e-5


def _rsqrt(a):
    a0 = a[0]
    i = lax.bitcast_convert_type(a0, jnp.int32)
    i = jnp.int32(0x5F3759DF) - lax.shift_right_logical(i, jnp.int32(1))
    y = jnp.broadcast_to(lax.bitcast_convert_type(i, jnp.float32), (16,))
    h = a * jnp.float32(-0.5)
    for _ in range(3):
        y = y * (h * (y * y) + jnp.float32(1.5))
    return y


def _lane_sum(v, perms):
    for p in perms:
        v = v + v.at[p].get(mode="promise_in_bounds")
    return v


def _body(aseq, sseq, atab, stab, ptab, gam, bet, out,
          aidx_v, sidx_v, pos_v, comb_v, g_v, b_v, bufd, semd):
    wid = lax.axis_index("s") * 2 + lax.axis_index("c")
    r0 = wid * _ROWS_PER_W

    pltpu.sync_copy(aseq.at[pl.ds(r0, _ROWS_PER_W)], aidx_v)
    pltpu.sync_copy(sseq.at[pl.ds(r0, _ROWS_PER_W)], sidx_v)
    pltpu.sync_copy(ptab.at[pl.ds(0, _L)], pos_v)
    pltpu.sync_copy(gam, g_v)
    pltpu.sync_copy(bet, b_v)
    stage = bufd.at[0]
    pltpu.sync_copy(atab, stage.at[pl.ds(0, _N_AMINO)])
    pltpu.sync_copy(stab, stage.at[pl.ds(32, _N_STRUCT)])

    def _build(ci, carry):
        ai = lax.shift_right_logical(ci, jnp.int32(3))
        si = lax.bitwise_and(ci, jnp.int32(_N_STRUCT - 1))
        for j in range(_NJ):
            a = stage[ai, pl.ds(j * 16, 16)]
            s = stage[si + 32, pl.ds(j * 16, 16)]
            comb_v[ci, pl.ds(j * 16, 16)] = a + s
        return carry

    lax.fori_loop(jnp.int32(0), jnp.int32(_N_AMINO * _N_STRUCT), _build, 0)

    idx16 = lax.iota(jnp.int32, 16)
    perms = [idx16 ^ jnp.int32(1 << k) for k in range(4)]
    g = [g_v[pl.ds(j * 16, 16)] for j in range(_NJ)]
    b = [b_v[pl.ds(j * 16, 16)] for j in range(_NJ)]
    inv_d = jnp.float32(1.0 / _D)

    def _row(r, carry):
        buf = bufd.at[0]

        @plsc.parallel_loop(jnp.int32(0), jnp.int32(_L), unroll=4)
        def _token(l):
            lbase = pl.multiple_of(lax.bitwise_and(l, jnp.int32(-16)), 16)
            k = lax.bitwise_and(l, jnp.int32(15))
            av = aidx_v[r, pl.ds(lbase, 16)]
            sv = sidx_v[r, pl.ds(lbase, 16)]
            civ = av * jnp.int32(_N_STRUCT) + sv
            rot = lax.bitwise_and(idx16 + jnp.broadcast_to(k, (16,)),
                                  jnp.int32(15))
            ci = civ.at[rot].get(mode="promise_in_bounds")[0]
            x = [comb_v[ci, pl.ds(j * 16, 16)] + pos_v[l, pl.ds(j * 16, 16)]
                 for j in range(_NJ)]
            s = ((x[0] + x[1]) + (x[2] + x[3])) + ((x[4] + x[5]) + (x[6] + x[7]))
            s = _lane_sum(s, perms)
            sq = [xj * xj for xj in x]
            q = ((sq[0] + sq[1]) + (sq[2] + sq[3])) + ((sq[4] + sq[5]) + (sq[6] + sq[7]))
            q = _lane_sum(q, perms)
            mean = s * inv_d
            var = q * inv_d - mean * mean
            rstd = _rsqrt(var + jnp.float32(_EPS))
            c = jnp.float32(0.0) - mean * rstd
            for j in range(_NJ):
                y = x[j] * rstd + c
                buf[l, pl.ds(j * 16, 16)] = y * g[j] + b[j]

        pltpu.sync_copy(buf, out.at[r0 + r])
        return carry

    lax.fori_loop(jnp.int32(0), jnp.int32(_ROWS_PER_W), _row, 0)


_sc_kernel = functools.partial(
    pl.kernel,
    out_type=jax.ShapeDtypeStruct((_B, _L, _D), jnp.float32),
    mesh=plsc.VectorSubcoreMesh(core_axis_name="c", subcore_axis_name="s"),
    scratch_types=[
        pltpu.VMEM((_ROWS_PER_W, _L), jnp.int32),
        pltpu.VMEM((_ROWS_PER_W, _L), jnp.int32),
        pltpu.VMEM((_L, _D), jnp.float32),
        pltpu.VMEM((_N_AMINO * _N_STRUCT, _D), jnp.float32),
        pltpu.VMEM((_D,), jnp.float32),
        pltpu.VMEM((_D,), jnp.float32),
        pltpu.VMEM((1, _L, _D), jnp.float32),
        pltpu.SemaphoreType.DMA((2,)),
    ],
)(_body)


def kernel(amino_seq, struct_seq, amino_table, struct_table, pos_table, gamma, beta):
    return _sc_kernel(amino_seq, struct_seq, amino_table, struct_table,
                      pos_table, gamma, beta)
